# baseline (speedup 1.0000x reference)
import functools

import jax
import jax.numpy as jnp
from jax import lax
from jax.experimental import pallas as pl
from jax.experimental.pallas import tpu as pltpu
from jax.experimental.pallas import tpu_sc as plsc

_NC = 2
_NS = 16
_NW = _NC * _NS
_L = 16

_CHUNK_ROWS = 4


def _sc_body(x_hbm, mins_hbm, flag_hbm, buf, out_v, flag_v, sem0, sem1,
             *, rows, cols, last_dims_equal):
    rows_per_w = rows // _NW
    nchunks = rows_per_w // _CHUNK_ROWS
    wid = lax.axis_index("s") * _NC + lax.axis_index("c")
    base = wid * rows_per_w
    sems = (sem0, sem1)

    def issue(c, slot):
        return pltpu.async_copy(
            x_hbm.at[pl.ds(base + c * _CHUNK_ROWS, _CHUNK_ROWS)],
            buf.at[slot], sems[slot])

    def lane_min_splat(v):
        for sh in (8, 4, 2, 1):
            idx = jnp.bitwise_xor(lax.iota(jnp.int32, _L), sh)
            v = jnp.minimum(v, jnp.take_along_axis(v, idx, axis=0))
        return v

    def reduce_chunk(c, slot):
        nacc = 4
        for r in range(_CHUNK_ROWS):
            def step(j, accs):
                return tuple(
                    jnp.minimum(accs[i],
                                buf[slot, r, pl.ds((j * nacc + i) * _L, _L)])
                    for i in range(nacc))
            accs = lax.fori_loop(
                0, cols // (nacc * _L), step,
                tuple(jnp.full((_L,), jnp.inf, jnp.float32)
                      for _ in range(nacc)),
                unroll=8)
            acc = jnp.minimum(jnp.minimum(accs[0], accs[1]),
                              jnp.minimum(accs[2], accs[3]))
            out_v[c * _CHUNK_ROWS + r, :] = lane_min_splat(acc)

    dsc = issue(0, 0)
    for c in range(nchunks):
        slot = c % 2
        dsc.wait()
        if c + 1 < nchunks:
            dsc = issue(c + 1, (c + 1) % 2)
        reduce_chunk(c, slot)

    pltpu.sync_copy(out_v, mins_hbm.at[pl.ds(base, rows_per_w)])

    @pl.when(wid == 0)
    def _():
        flag_v[...] = jnp.full((_L,), 1.0 if last_dims_equal else 0.0,
                               jnp.float32)
        pltpu.sync_copy(flag_v, flag_hbm)


def kernel(x):
    b0, b1, k = x.shape
    rows = b0 * b1
    xr = x.reshape(rows, k)
    mesh = plsc.VectorSubcoreMesh(core_axis_name="c", subcore_axis_name="s")
    body = functools.partial(_sc_body, rows=rows, cols=k,
                             last_dims_equal=(b1 == k))
    rows_per_w = rows // _NW
    mins, flag = pl.kernel(
        body,
        out_type=[
            jax.ShapeDtypeStruct((rows, _L), jnp.float32),
            jax.ShapeDtypeStruct((_L,), jnp.float32),
        ],
        mesh=mesh,
        scratch_types=[
            pltpu.VMEM((2, _CHUNK_ROWS, k), jnp.float32),
            pltpu.VMEM((rows_per_w, _L), jnp.float32),
            pltpu.VMEM((_L,), jnp.float32),
            pltpu.SemaphoreType.DMA,
            pltpu.SemaphoreType.DMA,
        ],
    )(xr)
    del mins
    return flag[0].astype(jnp.bool_)


# speedup vs baseline: 3.3748x; 3.3748x over previous
import functools

import jax
import jax.numpy as jnp
from jax.experimental import pallas as pl


_BLOCK_ROWS = 256


def _body(x_ref, mins_ref, flag_ref, *, last_dims_equal):
    mins_ref[...] = jnp.min(x_ref[...], axis=1)

    @pl.when(pl.program_id(0) == 0)
    def _():
        flag_ref[...] = jnp.full((1, 1), 1.0 if last_dims_equal else 0.0,
                                 jnp.float32)


def kernel(x):
    b0, b1, k = x.shape
    rows = b0 * b1
    xr = x.reshape(rows, k)
    body = functools.partial(_body, last_dims_equal=(b1 == k))
    mins, flag = pl.pallas_call(
        body,
        grid=(rows // _BLOCK_ROWS,),
        in_specs=[pl.BlockSpec((_BLOCK_ROWS, k), lambda i: (i, 0))],
        out_specs=[
            pl.BlockSpec((_BLOCK_ROWS,), lambda i: (i,)),
            pl.BlockSpec((1, 1), lambda i: (0, 0)),
        ],
        out_shape=[
            jax.ShapeDtypeStruct((rows,), jnp.float32),
            jax.ShapeDtypeStruct((1, 1), jnp.float32),
        ],
    )(xr)
    del mins
    return flag[0, 0].astype(jnp.bool_)
